# Initial kernel scaffold; baseline (speedup 1.0000x reference)
import jax
import jax.numpy as jnp
from jax.experimental import pallas as pl


def kernel(query, key, value, proj_dir):
    raise NotImplementedError("write your pallas kernel here")


# speedup vs baseline: 2.5378x; 2.5378x over previous
import functools
import math

import jax
import jax.numpy as jnp
import numpy as np
from jax.experimental import pallas as pl

_NUM_PROJS = 7
_BLOCK = 256
_SAMPLES = 256
_F32_MIN = float(np.finfo(np.float32).min)


def _gray_seq(size_n):
    if size_n == 1:
        return np.arange(2)
    a = _gray_seq(size_n - 1)
    return np.concatenate((a, a[::-1] + 2 ** (size_n - 1)))


def _take_rows(x, idx):
    return jnp.take_along_axis(x, idx[..., None], axis=2)


def _attn_body(qs_ref, ks_ref, vs_ref, ksub_ref, vsub_ref, samp_ref, out_ref,
               *, scale, num_blocks):
    j = pl.program_id(1)
    q = qs_ref[0]
    k = ks_ref[0]
    v = vs_ref[0]

    dot = functools.partial(
        jax.lax.dot_general,
        precision=jax.lax.Precision.HIGHEST,
        preferred_element_type=jnp.float32,
    )

    qk1 = dot(q, k, dimension_numbers=(((1,), (1,)), ((), ()))) * scale
    m1 = jnp.max(qk1, axis=-1)
    e1 = jnp.exp(qk1 - m1[:, None])
    s1 = jnp.sum(e1, axis=-1)
    a1 = dot(e1, v, dimension_numbers=(((1,), (0,)), ((), ()))) / s1[:, None]
    lse1 = m1 + jnp.log(s1)

    ksub = ksub_ref[0]
    vsub = vsub_ref[0]
    samp = samp_ref[0, 0]
    bias = jnp.where(samp // _BLOCK == j, _F32_MIN, 0.0).astype(jnp.float32)
    qk2 = dot(q, ksub, dimension_numbers=(((1,), (1,)), ((), ()))) * scale
    qk2 = qk2 + bias[None, :]
    m2 = jnp.max(qk2, axis=-1)
    e2 = jnp.exp(qk2 - m2[:, None])
    s2 = jnp.sum(e2, axis=-1)
    a2 = dot(e2, vsub, dimension_numbers=(((1,), (0,)), ((), ()))) / s2[:, None]
    lse2 = m2 + jnp.log(s2) + math.log(float(num_blocks))

    c = 1.0 / (1.0 + jnp.exp(lse2 - lse1))
    out_ref[0] = c[:, None] * a1 + (1.0 - c[:, None]) * a2


def kernel(query, key, value, proj_dir):
    b, h, n, d = query.shape
    bh = b * h
    num_blocks = n // _BLOCK
    scale = d ** (-0.5)

    perm = jnp.asarray(_gray_seq(_NUM_PROJS))
    enc_vec = (2 ** jnp.arange(_NUM_PROJS)).reshape(1, 1, 1, _NUM_PROJS)

    def lsh_hash(mat):
        mask = jnp.matmul(mat, proj_dir) > 0
        bin_ids = (mask * enc_vec).sum(-1)
        return perm[bin_ids]

    q_idx = jnp.argsort(lsh_hash(query), axis=2, stable=True)
    k_idx = jnp.argsort(lsh_hash(key), axis=2, stable=True)
    q_idx_inv = jnp.argsort(q_idx, axis=2, stable=True)

    query_sorted = _take_rows(query, q_idx)
    key_sorted = _take_rows(key, k_idx)
    value_sorted = _take_rows(value, k_idx)

    sampled_set = jax.random.randint(jax.random.key(42), (b, h, _SAMPLES), 0, n)
    key_subset = _take_rows(key_sorted, sampled_set)
    value_subset = _take_rows(value_sorted, sampled_set)

    qs = query_sorted.reshape(bh, n, d)
    ks = key_sorted.reshape(bh, n, d)
    vs = value_sorted.reshape(bh, n, d)
    ksub = key_subset.reshape(bh, _SAMPLES, d)
    vsub = value_subset.reshape(bh, _SAMPLES, d)
    samp = sampled_set.reshape(bh, 1, _SAMPLES).astype(jnp.int32)

    attn_sorted = pl.pallas_call(
        functools.partial(_attn_body, scale=scale, num_blocks=num_blocks),
        grid=(bh, num_blocks),
        in_specs=[
            pl.BlockSpec((1, _BLOCK, d), lambda i, j: (i, j, 0)),
            pl.BlockSpec((1, _BLOCK, d), lambda i, j: (i, j, 0)),
            pl.BlockSpec((1, _BLOCK, d), lambda i, j: (i, j, 0)),
            pl.BlockSpec((1, _SAMPLES, d), lambda i, j: (i, 0, 0)),
            pl.BlockSpec((1, _SAMPLES, d), lambda i, j: (i, 0, 0)),
            pl.BlockSpec((1, 1, _SAMPLES), lambda i, j: (i, 0, 0)),
        ],
        out_specs=pl.BlockSpec((1, _BLOCK, d), lambda i, j: (i, j, 0)),
        out_shape=jax.ShapeDtypeStruct((bh, n, d), jnp.float32),
    )(qs, ks, vs, ksub, vsub, samp)

    attn_sorted = attn_sorted.reshape(b, h, n, d)
    return _take_rows(attn_sorted, q_idx_inv)
